# baseline (speedup 1.0000x reference)
import functools
import math

import jax
import jax.numpy as jnp
from jax import lax
from jax.experimental import pallas as pl
from jax.experimental.pallas import tpu as pltpu
from jax.experimental.pallas import tpu_sc as plsc

D_MODEL = 64
SCALE = math.sqrt(D_MODEL)

NC, NS, L = 2, 16, 16
NW = NC * NS

IDXW = 128
C = 512
NSUB = C // IDXW


def _make_kernel(B, V):
    b_per_w = B // NW
    G = b_per_w // C
    mesh = plsc.VectorSubcoreMesh(core_axis_name="c", subcore_axis_name="s")

    @functools.partial(
        pl.kernel,
        out_type=jax.ShapeDtypeStruct((B, D_MODEL), jnp.float32),
        mesh=mesh,
        scratch_types=[
            pltpu.VMEM((NSUB, IDXW), jnp.int32),
            pltpu.VMEM((C, D_MODEL), jnp.float32),
            pltpu.SemaphoreType.DMA,
        ],
        compiler_params=pltpu.CompilerParams(use_tc_tiling_on_sc=False),
    )
    def emb_kernel(x_hbm, w_hbm, out_hbm, idx_v, rows_v, sem):
        wid = lax.axis_index("s") * NC + lax.axis_index("c")
        base = wid * b_per_w
        xrow0 = wid * (b_per_w // IDXW)

        def group(g, carry):
            pltpu.sync_copy(x_hbm.at[pl.ds(xrow0 + g * NSUB, NSUB)], idx_v)
            copies = [
                pltpu.async_copy(
                    w_hbm.at[idx_v.at[j]],
                    rows_v.at[pl.ds(j * IDXW, IDXW)],
                    sem,
                )
                for j in range(NSUB)
            ]
            for cp in copies:
                cp.wait()

            def srow(r, c2):
                for k in range(D_MODEL // L):
                    rows_v[r, pl.ds(k * L, L)] = rows_v[r, pl.ds(k * L, L)] * SCALE
                return c2

            lax.fori_loop(0, C, srow, 0)

            pltpu.sync_copy(rows_v, out_hbm.at[pl.ds(base + g * C, C)])
            return carry

        lax.fori_loop(0, G, group, 0)

    return emb_kernel


def kernel(x, W):
    B0, S = x.shape
    B = B0 * S
    V = W.shape[0]
    x2d = x.reshape(B // IDXW, IDXW).astype(jnp.int32)
    out = _make_kernel(B, V)(x2d, W)
    return out.reshape(B0, S, D_MODEL)


# speedup vs baseline: 1.1350x; 1.1350x over previous
import functools
import math

import jax
import jax.numpy as jnp
from jax import lax
from jax.experimental import pallas as pl
from jax.experimental.pallas import tpu as pltpu
from jax.experimental.pallas import tpu_sc as plsc

D_MODEL = 64
SCALE = math.sqrt(D_MODEL)

NC, NS, L = 2, 16, 16
NW = NC * NS

IDXW = 128
C = 256
NSUB = C // IDXW
NBUF = 4
ROW_UNROLL = 8


def _make_kernel(B, V):
    b_per_w = B // NW
    G = b_per_w // C
    assert G % NBUF == 0
    mesh = plsc.VectorSubcoreMesh(core_axis_name="c", subcore_axis_name="s")

    scratch = [pltpu.VMEM((b_per_w // IDXW, IDXW), jnp.int32)]
    scratch += [pltpu.VMEM((C, D_MODEL), jnp.float32) for _ in range(NBUF)]
    scratch += [pltpu.SemaphoreType.DMA for _ in range(2 * NBUF)]

    @functools.partial(
        pl.kernel,
        out_type=jax.ShapeDtypeStruct((B, D_MODEL), jnp.float32),
        mesh=mesh,
        scratch_types=scratch,
        compiler_params=pltpu.CompilerParams(use_tc_tiling_on_sc=False),
    )
    def emb_kernel(x_hbm, w_hbm, out_hbm, idx_v, *bufs_and_sems):
        rows = bufs_and_sems[:NBUF]
        gsem = bufs_and_sems[NBUF:2 * NBUF]
        ssem = bufs_and_sems[2 * NBUF:]

        wid = lax.axis_index("s") * NC + lax.axis_index("c")
        base = wid * b_per_w
        xrow0 = wid * (b_per_w // IDXW)

        pltpu.sync_copy(x_hbm.at[pl.ds(xrow0, b_per_w // IDXW)], idx_v)

        def fire_gather(gg, b):
            for j in range(NSUB):
                pltpu.async_copy(
                    w_hbm.at[idx_v.at[gg * NSUB + j]],
                    rows[b].at[pl.ds(j * IDXW, IDXW)],
                    gsem[b],
                )

        def drain_gather(b):
            for j in range(NSUB):
                pltpu.make_async_copy(
                    w_hbm.at[idx_v.at[j]],
                    rows[b].at[pl.ds(j * IDXW, IDXW)],
                    gsem[b],
                ).wait()

        fire_gather(0, 0)

        def outer(g0, carry):
            for b in range(NBUF):
                gg = g0 * NBUF + b
                nb = (b + 1) % NBUF

                @pl.when(gg >= NBUF - 1)
                def _():
                    pltpu.make_async_copy(
                        rows[nb], out_hbm.at[pl.ds(0, C)], ssem[nb]
                    ).wait()

                @pl.when(gg + 1 < G)
                def _():
                    fire_gather(gg + 1, nb)

                drain_gather(b)

                buf = rows[b]

                @plsc.parallel_loop(0, C, unroll=ROW_UNROLL)
                def _(r):
                    for k in range(D_MODEL // L):
                        buf[r, pl.ds(k * L, L)] = buf[r, pl.ds(k * L, L)] * SCALE

                pltpu.async_copy(
                    buf, out_hbm.at[pl.ds(base + gg * C, C)], ssem[b]
                )
            return carry

        lax.fori_loop(0, G // NBUF, outer, 0)

        for b in range(1, NBUF):
            pltpu.make_async_copy(
                rows[b], out_hbm.at[pl.ds(0, C)], ssem[b]
            ).wait()

    return emb_kernel


def kernel(x, W):
    B0, S = x.shape
    B = B0 * S
    V = W.shape[0]
    x2d = x.reshape(B // IDXW, IDXW).astype(jnp.int32)
    out = _make_kernel(B, V)(x2d, W)
    return out.reshape(B0, S, D_MODEL)


# speedup vs baseline: 1.1386x; 1.0031x over previous
import functools
import math

import jax
import jax.numpy as jnp
from jax import lax
from jax.experimental import pallas as pl
from jax.experimental.pallas import tpu as pltpu
from jax.experimental.pallas import tpu_sc as plsc

D_MODEL = 64
SCALE = math.sqrt(D_MODEL)

NC, NS, L = 2, 16, 16
NW = NC * NS

IDXW = 128
C = 256
NSUB = C // IDXW
NBUF = 4
ROW_UNROLL = 8


def _make_kernel(B, V):
    b_per_w = B // NW
    G = b_per_w // C
    assert G % NBUF == 0
    mesh = plsc.VectorSubcoreMesh(core_axis_name="c", subcore_axis_name="s")

    scratch = [pltpu.VMEM((b_per_w // IDXW, IDXW), jnp.int32)]
    scratch += [pltpu.VMEM((C, D_MODEL), jnp.float32) for _ in range(NBUF)]
    scratch += [pltpu.SemaphoreType.DMA for _ in range(2 * NBUF)]

    @functools.partial(
        pl.kernel,
        out_type=jax.ShapeDtypeStruct((B, D_MODEL), jnp.float32),
        mesh=mesh,
        scratch_types=scratch,
        compiler_params=pltpu.CompilerParams(
            use_tc_tiling_on_sc=False, skip_device_barrier=True
        ),
    )
    def emb_kernel(x_hbm, w_hbm, out_hbm, idx_v, *bufs_and_sems):
        rows = bufs_and_sems[:NBUF]
        gsem = bufs_and_sems[NBUF:2 * NBUF]
        ssem = bufs_and_sems[2 * NBUF:]

        wid = lax.axis_index("s") * NC + lax.axis_index("c")
        base = wid * b_per_w
        xrow0 = wid * (b_per_w // IDXW)

        pltpu.sync_copy(x_hbm.at[pl.ds(xrow0, b_per_w // IDXW)], idx_v)

        def fire_gather(gg, b):
            for j in range(NSUB):
                pltpu.async_copy(
                    w_hbm.at[idx_v.at[gg * NSUB + j]],
                    rows[b].at[pl.ds(j * IDXW, IDXW)],
                    gsem[b],
                )

        def drain_gather(b):
            for j in range(NSUB):
                pltpu.make_async_copy(
                    w_hbm.at[idx_v.at[j]],
                    rows[b].at[pl.ds(j * IDXW, IDXW)],
                    gsem[b],
                ).wait()

        fire_gather(0, 0)

        def outer(g0, carry):
            for b in range(NBUF):
                gg = g0 * NBUF + b
                nb = (b + 1) % NBUF

                @pl.when(gg >= NBUF - 1)
                def _():
                    pltpu.make_async_copy(
                        rows[nb], out_hbm.at[pl.ds(0, C)], ssem[nb]
                    ).wait()

                @pl.when(gg + 1 < G)
                def _():
                    fire_gather(gg + 1, nb)

                drain_gather(b)

                buf = rows[b]

                @plsc.parallel_loop(0, C, unroll=ROW_UNROLL)
                def _(r):
                    for k in range(D_MODEL // L):
                        buf[r, pl.ds(k * L, L)] = buf[r, pl.ds(k * L, L)] * SCALE

                pltpu.async_copy(
                    buf, out_hbm.at[pl.ds(base + gg * C, C)], ssem[b]
                )
            return carry

        lax.fori_loop(0, G // NBUF, outer, 0)

        for b in range(1, NBUF):
            pltpu.make_async_copy(
                rows[b], out_hbm.at[pl.ds(0, C)], ssem[b]
            ).wait()

    return emb_kernel


def kernel(x, W):
    B0, S = x.shape
    B = B0 * S
    V = W.shape[0]
    x2d = x.reshape(B // IDXW, IDXW).astype(jnp.int32)
    out = _make_kernel(B, V)(x2d, W)
    return out.reshape(B0, S, D_MODEL)
